# speedup vs baseline: 1.0851x; 1.0851x over previous
import functools

import jax
import jax.numpy as jnp
from jax.experimental import pallas as pl

N = 16384
IN_PLANES = 128
HIDDEN = 128
NUM_HEADS = 4
HEAD_DIM = HIDDEN // NUM_HEADS
STRIDE = 4
NSAMPLE = 16
M = N // STRIDE

KNN_BLOCK = 128
ATT_BLOCK = 256


def _grid_sampling(xyz, num_samples):
    n = xyz.shape[0]
    vmin = xyz.min(axis=0)
    vmax = xyz.max(axis=0)
    voxel_size = (vmax - vmin) / (num_samples ** (1.0 / 3.0))
    voxel_size = voxel_size / 2.0
    grid = jnp.floor((xyz - vmin) / voxel_size).astype(jnp.int32)
    K = jnp.max(grid) + 1
    code = (grid[:, 0] * K + grid[:, 1]) * K + grid[:, 2]
    order = jnp.argsort(code, stable=True)
    sorted_code = code[order]
    first = jnp.concatenate(
        [jnp.ones((1,), dtype=bool), sorted_code[1:] != sorted_code[:-1]]
    )
    is_rep = jnp.zeros((n,), dtype=bool).at[order].set(first)
    idx_all = jnp.arange(n, dtype=jnp.int32)
    rank = jnp.where(is_rep, idx_all, n + idx_all)
    final = jnp.argsort(rank, stable=True)[:num_samples]
    return final


def _knn_body(np_ref, pT_ref, out_ref):
    q = np_ref[...]
    pT = pT_ref[...]
    psq = jnp.sum(pT * pT, axis=0, keepdims=True)
    scores = psq - 2.0 * jnp.dot(q, pT,
                                 preferred_element_type=jnp.float32)
    col = jax.lax.broadcasted_iota(jnp.int32, scores.shape, 1)
    d = scores
    cols = []
    for _ in range(NSAMPLE):
        m = jnp.min(d, axis=1, keepdims=True)
        sel = jnp.where(d == m, col, N)
        ij = jnp.min(sel, axis=1, keepdims=True)
        cols.append(ij)
        d = jnp.where(col == ij, jnp.inf, d)
    out_ref[...] = jnp.concatenate(cols, axis=1)


def _knn(n_p, pT):
    grid = (M // KNN_BLOCK,)
    return pl.pallas_call(
        _knn_body,
        grid=grid,
        in_specs=[
            pl.BlockSpec((KNN_BLOCK, 3), lambda i: (i, 0)),
            pl.BlockSpec((3, N), lambda i: (0, 0)),
        ],
        out_specs=pl.BlockSpec((KNN_BLOCK, NSAMPLE), lambda i: (i, 0)),
        out_shape=jax.ShapeDtypeStruct((M, NSAMPLE), jnp.int32),
    )(n_p, pT)


def _angle_parts(ax, ay, az, bx, by, bz):
    cx = ay * bz - az * by
    cy = az * bx - ax * bz
    cz = ax * by - ay * bx
    cn = jnp.sqrt(cx * cx + cy * cy + cz * cz + 1e-12)
    dot = ax * bx + ay * by + az * bz
    return jnp.arctan2(cn, dot)


def _attn_body(npos_ref, nnorm_ref, cpx_ref, cpy_ref, cpz_ref,
               cnx_ref, cny_ref, cnz_ref, xq_ref, cx_ref,
               Wq_ref, Wk_ref, Wv_ref, Wp1_ref, Wp2_ref, Wo_ref, out_ref):
    B = ATT_BLOCK
    npos = npos_ref[...]
    nnorm = nnorm_ref[...]
    qpx, qpy, qpz = npos[:, 0:1], npos[:, 1:2], npos[:, 2:3]
    nx, ny, nz = nnorm[:, 0:1], nnorm[:, 1:2], nnorm[:, 2:3]
    cpx, cpy, cpz = cpx_ref[...], cpy_ref[...], cpz_ref[...]
    cnx, cny, cnz = cnx_ref[...], cny_ref[...], cnz_ref[...]

    dx, dy, dz = cpx - qpx, cpy - qpy, cpz - qpz
    dist = jnp.sqrt(dx * dx + dy * dy + dz * dz + 1e-12)
    f1 = _angle_parts(nx, ny, nz, dx, dy, dz)
    f2 = _angle_parts(cnx, cny, cnz, dx, dy, dz)
    f3 = _angle_parts(nx, ny, nz, cnx, cny, cnz)

    Wp1 = Wp1_ref[...]
    pe3 = (f1[:, :, None] * Wp1[0:1, :][None]
           + f2[:, :, None] * Wp1[1:2, :][None]
           + f3[:, :, None] * Wp1[2:3, :][None]
           + dist[:, :, None] * Wp1[3:4, :][None])
    pe2 = jnp.maximum(pe3, 0.0).reshape(B * NSAMPLE, HIDDEN) @ Wp2_ref[...]

    cx = cx_ref[...]
    k2 = cx @ Wk_ref[...] + pe2
    v2 = cx @ Wv_ref[...] + pe2
    q2 = xq_ref[...] @ Wq_ref[...]

    k3 = k2.reshape(B, NSAMPLE, HIDDEN)
    v3 = v2.reshape(B, NSAMPLE, HIDDEN)
    prod = q2[:, None, :] * k3
    scale = 1.0 / jnp.sqrt(jnp.float32(HEAD_DIM))
    outs = []
    for h in range(NUM_HEADS):
        sl = slice(h * HEAD_DIM, (h + 1) * HEAD_DIM)
        lh = jnp.sum(prod[:, :, sl], axis=2) * scale
        mh = jnp.max(lh, axis=1, keepdims=True)
        e = jnp.exp(lh - mh)
        a = e / jnp.sum(e, axis=1, keepdims=True)
        outs.append(jnp.sum(a[:, :, None] * v3[:, :, sl], axis=1))
    out = jnp.concatenate(outs, axis=1)
    out_ref[...] = out @ Wo_ref[...]


def _attention(npos, nnorm, cpx, cpy, cpz, cnx, cny, cnz, xq, cx,
               Wq, Wk, Wv, Wp1, Wp2, Wo):
    B = ATT_BLOCK
    grid = (M // B,)
    bs = lambda r: pl.BlockSpec((B, r), lambda i: (i, 0))
    full = lambda a, b: pl.BlockSpec((a, b), lambda i: (0, 0))
    return pl.pallas_call(
        _attn_body,
        grid=grid,
        in_specs=[
            bs(3), bs(3),
            bs(NSAMPLE), bs(NSAMPLE), bs(NSAMPLE),
            bs(NSAMPLE), bs(NSAMPLE), bs(NSAMPLE),
            bs(IN_PLANES),
            pl.BlockSpec((B * NSAMPLE, IN_PLANES), lambda i: (i, 0)),
            full(IN_PLANES, HIDDEN), full(IN_PLANES, HIDDEN),
            full(IN_PLANES, HIDDEN), full(4, HIDDEN),
            full(HIDDEN, HIDDEN), full(HIDDEN, HIDDEN),
        ],
        out_specs=pl.BlockSpec((B, HIDDEN), lambda i: (i, 0)),
        out_shape=jax.ShapeDtypeStruct((M, HIDDEN), jnp.float32),
    )(npos, nnorm, cpx, cpy, cpz, cnx, cny, cnz, xq, cx,
      Wq, Wk, Wv, Wp1, Wp2, Wo)


def kernel(p, x, o, n, Wq, Wk, Wv, Wp1, Wp2, Wo):
    idx = _grid_sampling(p, M).astype(jnp.int32)
    n_p = p[idx]
    n_n = n[idx]
    n_o = (o // STRIDE).astype(jnp.int32)

    group_idx = _knn(n_p, p.T)

    gi = group_idx.reshape(-1)
    cpx = jnp.take(p[:, 0], gi, axis=0).reshape(M, NSAMPLE)
    cpy = jnp.take(p[:, 1], gi, axis=0).reshape(M, NSAMPLE)
    cpz = jnp.take(p[:, 2], gi, axis=0).reshape(M, NSAMPLE)
    cnx = jnp.take(n[:, 0], gi, axis=0).reshape(M, NSAMPLE)
    cny = jnp.take(n[:, 1], gi, axis=0).reshape(M, NSAMPLE)
    cnz = jnp.take(n[:, 2], gi, axis=0).reshape(M, NSAMPLE)
    cx = jnp.take(x, gi, axis=0)
    xq = x[idx]

    x_out = _attention(n_p, n_n, cpx, cpy, cpz, cnx, cny, cnz, xq, cx,
                       Wq, Wk, Wv, Wp1, Wp2, Wo)
    return (n_p, x_out, n_o, n_n, idx)


# speedup vs baseline: 3.7265x; 3.4343x over previous
import jax
import jax.numpy as jnp
from jax import lax
from jax.experimental import pallas as pl
from jax.experimental.pallas import tpu as pltpu
from jax.experimental.pallas import tpu_sc as plsc

N = 16384
IN_PLANES = 128
HIDDEN = 128
NUM_HEADS = 4
HEAD_DIM = HIDDEN // NUM_HEADS
STRIDE = 4
NSAMPLE = 16
M = N // STRIDE

KNN_BLOCK = 128
ATT_BLOCK = 256

SC_WORKERS = 32
CODE_TBL = 262144


def _grid_sampling(xyz, num_samples):
    n = xyz.shape[0]
    vmin = xyz.min(axis=0)
    vmax = xyz.max(axis=0)
    voxel_size = (vmax - vmin) / (num_samples ** (1.0 / 3.0))
    voxel_size = voxel_size / 2.0
    grid = jnp.floor((xyz - vmin) / voxel_size).astype(jnp.int32)
    K = jnp.max(grid) + 1
    code = (grid[:, 0] * K + grid[:, 1]) * K + grid[:, 2]
    iota = jnp.arange(n, dtype=jnp.int32)
    table = jnp.full((CODE_TBL,), n, jnp.int32).at[code].min(iota)
    is_rep = table[code] == iota
    cr = jnp.cumsum(is_rep.astype(jnp.int32))
    r = cr[-1]
    pos = jnp.where(is_rep, cr - 1, r + (iota - cr))
    out = jnp.zeros((n,), jnp.int32).at[pos].set(iota, unique_indices=True)
    return out[:num_samples]


def _knn_body(np_ref, pT_ref, out_ref):
    q = np_ref[...]
    pT = pT_ref[...]
    psq = jnp.sum(pT * pT, axis=0, keepdims=True)
    scores = psq - 2.0 * jnp.dot(q, pT,
                                 preferred_element_type=jnp.float32)
    col = jax.lax.broadcasted_iota(jnp.int32, scores.shape, 1)
    d = scores
    cols = []
    for _ in range(NSAMPLE):
        m = jnp.min(d, axis=1, keepdims=True)
        sel = jnp.where(d == m, col, N)
        ij = jnp.min(sel, axis=1, keepdims=True)
        cols.append(ij)
        d = jnp.where(col == ij, jnp.inf, d)
    out_ref[...] = jnp.concatenate(cols, axis=1)


def _knn(n_p, pT):
    return pl.pallas_call(
        _knn_body,
        grid=(M // KNN_BLOCK,),
        in_specs=[
            pl.BlockSpec((KNN_BLOCK, 3), lambda i: (i, 0)),
            pl.BlockSpec((3, N), lambda i: (0, 0)),
        ],
        out_specs=pl.BlockSpec((KNN_BLOCK, NSAMPLE), lambda i: (i, 0)),
        out_shape=jax.ShapeDtypeStruct((M, NSAMPLE), jnp.int32),
    )(n_p, pT)


def _pick_chunk(per_w, cap):
    if per_w <= cap:
        return per_w
    c = cap - cap % 16
    while per_w % c:
        c -= 16
    return c


def _make_sc_gather_cols(num_tables, total):
    per_w = total // SC_WORKERS
    chunk = _pick_chunk(per_w, 2048)
    n_chunks = per_w // chunk
    mesh = plsc.VectorSubcoreMesh(core_axis_name="c", subcore_axis_name="s")

    def body(*refs):
        tbl_hbm = refs[:num_tables]
        idx_hbm = refs[num_tables]
        out_hbm = refs[num_tables + 1:2 * num_tables + 1]
        tbl_v = refs[2 * num_tables + 1:3 * num_tables + 1]
        idx_v = refs[3 * num_tables + 1]
        out_v = refs[3 * num_tables + 2:]
        wid = lax.axis_index("s") * 2 + lax.axis_index("c")
        base = wid * per_w
        for t in range(num_tables):
            pltpu.sync_copy(tbl_hbm[t], tbl_v[t])

        def outer(ci, carry):
            off = base + ci * chunk
            pltpu.sync_copy(idx_hbm.at[pl.ds(off, chunk)], idx_v)

            def inner(k, c2):
                iv = idx_v[pl.ds(k * 16, 16)]
                for t in range(num_tables):
                    out_v[t][pl.ds(k * 16, 16)] = plsc.load_gather(
                        tbl_v[t], [iv])
                return c2

            lax.fori_loop(0, chunk // 16, inner, 0)
            for t in range(num_tables):
                pltpu.sync_copy(out_v[t], out_hbm[t].at[pl.ds(off, chunk)])
            return carry

        lax.fori_loop(0, n_chunks, outer, 0)

    return pl.kernel(
        body,
        out_type=[jax.ShapeDtypeStruct((total,), jnp.float32)] * num_tables,
        mesh=mesh,
        compiler_params=pltpu.CompilerParams(needs_layout_passes=False),
        scratch_types=(
            [pltpu.VMEM((N,), jnp.float32)] * num_tables
            + [pltpu.VMEM((chunk,), jnp.int32)]
            + [pltpu.VMEM((chunk,), jnp.float32)] * num_tables
        ),
    )


def _make_sc_gather_rows(total, depth):
    per_w = total // SC_WORKERS
    chunk = _pick_chunk(per_w, 128)
    n_chunks = per_w // chunk
    mesh = plsc.VectorSubcoreMesh(core_axis_name="c", subcore_axis_name="s")

    def body(tbl_hbm, idx_hbm, out_hbm, idx_v, rows_v, sem):
        wid = lax.axis_index("s") * 2 + lax.axis_index("c")
        base = wid * per_w

        def outer(ci, carry):
            off = base + ci * chunk
            pltpu.sync_copy(idx_hbm.at[pl.ds(off, chunk)], idx_v)
            pltpu.async_copy(tbl_hbm.at[idx_v], rows_v, sem).wait()
            pltpu.sync_copy(rows_v, out_hbm.at[pl.ds(off, chunk)])
            return carry

        lax.fori_loop(0, n_chunks, outer, 0)

    return pl.kernel(
        body,
        out_type=jax.ShapeDtypeStruct((total, depth), jnp.float32),
        mesh=mesh,
        compiler_params=pltpu.CompilerParams(needs_layout_passes=False),
        scratch_types=[
            pltpu.VMEM((chunk,), jnp.int32),
            pltpu.VMEM((chunk, depth), jnp.float32),
            pltpu.SemaphoreType.DMA,
        ],
    )


def _angle_parts(ax, ay, az, bx, by, bz):
    cx = ay * bz - az * by
    cy = az * bx - ax * bz
    cz = ax * by - ay * bx
    cn = jnp.sqrt(cx * cx + cy * cy + cz * cz + 1e-12)
    dot = ax * bx + ay * by + az * bz
    return jnp.arctan2(cn, dot)


def _attn_body(npos_ref, nnorm_ref, cpx_ref, cpy_ref, cpz_ref,
               cnx_ref, cny_ref, cnz_ref, xq_ref, cx_ref,
               Wq_ref, Wk_ref, Wv_ref, Wp1_ref, Wp2_ref, Wo_ref, out_ref):
    B = ATT_BLOCK
    npos = npos_ref[...]
    nnorm = nnorm_ref[...]
    qpx, qpy, qpz = npos[:, 0:1], npos[:, 1:2], npos[:, 2:3]
    nx, ny, nz = nnorm[:, 0:1], nnorm[:, 1:2], nnorm[:, 2:3]
    cpx, cpy, cpz = cpx_ref[...], cpy_ref[...], cpz_ref[...]
    cnx, cny, cnz = cnx_ref[...], cny_ref[...], cnz_ref[...]

    dx, dy, dz = cpx - qpx, cpy - qpy, cpz - qpz
    dist = jnp.sqrt(dx * dx + dy * dy + dz * dz + 1e-12)
    f1 = _angle_parts(nx, ny, nz, dx, dy, dz)
    f2 = _angle_parts(cnx, cny, cnz, dx, dy, dz)
    f3 = _angle_parts(nx, ny, nz, cnx, cny, cnz)

    Wp1 = Wp1_ref[...]
    pe3 = (f1[:, :, None] * Wp1[0:1, :][None]
           + f2[:, :, None] * Wp1[1:2, :][None]
           + f3[:, :, None] * Wp1[2:3, :][None]
           + dist[:, :, None] * Wp1[3:4, :][None])
    pe2 = jnp.maximum(pe3, 0.0).reshape(B * NSAMPLE, HIDDEN) @ Wp2_ref[...]

    cx = cx_ref[...]
    k2 = cx @ Wk_ref[...] + pe2
    v2 = cx @ Wv_ref[...] + pe2
    q2 = xq_ref[...] @ Wq_ref[...]

    k3 = k2.reshape(B, NSAMPLE, HIDDEN)
    v3 = v2.reshape(B, NSAMPLE, HIDDEN)
    prod = q2[:, None, :] * k3
    scale = 1.0 / jnp.sqrt(jnp.float32(HEAD_DIM))
    outs = []
    for h in range(NUM_HEADS):
        sl = slice(h * HEAD_DIM, (h + 1) * HEAD_DIM)
        lh = jnp.sum(prod[:, :, sl], axis=2) * scale
        mh = jnp.max(lh, axis=1, keepdims=True)
        e = jnp.exp(lh - mh)
        a = e / jnp.sum(e, axis=1, keepdims=True)
        outs.append(jnp.sum(a[:, :, None] * v3[:, :, sl], axis=1))
    out = jnp.concatenate(outs, axis=1)
    out_ref[...] = out @ Wo_ref[...]


def _attention(npos, nnorm, cpx, cpy, cpz, cnx, cny, cnz, xq, cx,
               Wq, Wk, Wv, Wp1, Wp2, Wo):
    B = ATT_BLOCK
    bs = lambda r: pl.BlockSpec((B, r), lambda i: (i, 0))
    full = lambda a, b: pl.BlockSpec((a, b), lambda i: (0, 0))
    return pl.pallas_call(
        _attn_body,
        grid=(M // B,),
        in_specs=[
            bs(3), bs(3),
            bs(NSAMPLE), bs(NSAMPLE), bs(NSAMPLE),
            bs(NSAMPLE), bs(NSAMPLE), bs(NSAMPLE),
            bs(IN_PLANES),
            pl.BlockSpec((B * NSAMPLE, IN_PLANES), lambda i: (i, 0)),
            full(IN_PLANES, HIDDEN), full(IN_PLANES, HIDDEN),
            full(IN_PLANES, HIDDEN), full(4, HIDDEN),
            full(HIDDEN, HIDDEN), full(HIDDEN, HIDDEN),
        ],
        out_specs=pl.BlockSpec((B, HIDDEN), lambda i: (i, 0)),
        out_shape=jax.ShapeDtypeStruct((M, HIDDEN), jnp.float32),
    )(npos, nnorm, cpx, cpy, cpz, cnx, cny, cnz, xq, cx,
      Wq, Wk, Wv, Wp1, Wp2, Wo)


def kernel(p, x, o, n, Wq, Wk, Wv, Wp1, Wp2, Wo):
    idx = _grid_sampling(p, M).astype(jnp.int32)
    n_p = p[idx]
    n_n = n[idx]
    n_o = (o // STRIDE).astype(jnp.int32)

    group_idx = _knn(n_p, p.T)
    gi = group_idx.reshape(-1)

    g6 = _make_sc_gather_cols(6, M * NSAMPLE)
    gpx, gpy, gpz, gnx, gny, gnz = g6(
        p[:, 0], p[:, 1], p[:, 2], n[:, 0], n[:, 1], n[:, 2], gi)
    cpx = gpx.reshape(M, NSAMPLE)
    cpy = gpy.reshape(M, NSAMPLE)
    cpz = gpz.reshape(M, NSAMPLE)
    cnx = gnx.reshape(M, NSAMPLE)
    cny = gny.reshape(M, NSAMPLE)
    cnz = gnz.reshape(M, NSAMPLE)

    gi_all = jnp.concatenate([gi, idx])
    rows = _make_sc_gather_rows(M * NSAMPLE + M, IN_PLANES)(x, gi_all)
    cx = rows[:M * NSAMPLE]
    xq = rows[M * NSAMPLE:]

    x_out = _attention(n_p, n_n, cpx, cpy, cpz, cnx, cny, cnz, xq, cx,
                       Wq, Wk, Wv, Wp1, Wp2, Wo)
    return (n_p, x_out, n_o, n_n, idx)


# speedup vs baseline: 3.7468x; 1.0055x over previous
import jax
import jax.numpy as jnp
from jax import lax
from jax.experimental import pallas as pl
from jax.experimental.pallas import tpu as pltpu
from jax.experimental.pallas import tpu_sc as plsc

N = 16384
IN_PLANES = 128
HIDDEN = 128
NUM_HEADS = 4
HEAD_DIM = HIDDEN // NUM_HEADS
STRIDE = 4
NSAMPLE = 16
M = N // STRIDE

KNN_BLOCK = 128
ATT_BLOCK = 512

SC_WORKERS = 32
CODE_TBL = 262144


def _grid_sampling(xyz, num_samples):
    n = xyz.shape[0]
    vmin = xyz.min(axis=0)
    vmax = xyz.max(axis=0)
    voxel_size = (vmax - vmin) / (num_samples ** (1.0 / 3.0))
    voxel_size = voxel_size / 2.0
    grid = jnp.floor((xyz - vmin) / voxel_size).astype(jnp.int32)
    K = jnp.max(grid) + 1
    code = (grid[:, 0] * K + grid[:, 1]) * K + grid[:, 2]
    iota = jnp.arange(n, dtype=jnp.int32)
    table = jnp.full((CODE_TBL,), n, jnp.int32).at[code].min(iota)
    is_rep = table[code] == iota
    cr = jnp.cumsum(is_rep.astype(jnp.int32))
    r = cr[-1]
    pos = jnp.where(is_rep, cr - 1, r + (iota - cr))
    out = jnp.zeros((n,), jnp.int32).at[pos].set(iota, unique_indices=True)
    return out[:num_samples]


def _knn_body(np_ref, pT_ref, out_ref):
    q = np_ref[...]
    pT = pT_ref[...]
    psq = jnp.sum(pT * pT, axis=0, keepdims=True)
    scores = psq - 2.0 * jnp.dot(q, pT,
                                 preferred_element_type=jnp.float32)
    col = jax.lax.broadcasted_iota(jnp.int32, scores.shape, 1)
    d = scores
    cols = []
    for _ in range(NSAMPLE):
        m = jnp.min(d, axis=1, keepdims=True)
        sel = jnp.where(d == m, col, N)
        ij = jnp.min(sel, axis=1, keepdims=True)
        cols.append(ij)
        d = jnp.where(col == ij, jnp.inf, d)
    out_ref[...] = jnp.concatenate(cols, axis=1)


def _knn(n_p, pT):
    return pl.pallas_call(
        _knn_body,
        grid=(M // KNN_BLOCK,),
        in_specs=[
            pl.BlockSpec((KNN_BLOCK, 3), lambda i: (i, 0)),
            pl.BlockSpec((3, N), lambda i: (0, 0)),
        ],
        out_specs=pl.BlockSpec((KNN_BLOCK, NSAMPLE), lambda i: (i, 0)),
        out_shape=jax.ShapeDtypeStruct((M, NSAMPLE), jnp.int32),
    )(n_p, pT)


def _pick_chunk(per_w, cap):
    if per_w <= cap:
        return per_w
    c = cap - cap % 16
    while per_w % c:
        c -= 16
    return c


def _make_sc_gather_cols(num_tables, total):
    per_w = total // SC_WORKERS
    chunk = _pick_chunk(per_w, 2048)
    n_chunks = per_w // chunk
    mesh = plsc.VectorSubcoreMesh(core_axis_name="c", subcore_axis_name="s")

    def body(*refs):
        tbl_hbm = refs[:num_tables]
        idx_hbm = refs[num_tables]
        out_hbm = refs[num_tables + 1:2 * num_tables + 1]
        tbl_v = refs[2 * num_tables + 1:3 * num_tables + 1]
        idx_v = refs[3 * num_tables + 1]
        out_v = refs[3 * num_tables + 2:]
        wid = lax.axis_index("s") * 2 + lax.axis_index("c")
        base = wid * per_w
        for t in range(num_tables):
            pltpu.sync_copy(tbl_hbm[t], tbl_v[t])

        def outer(ci, carry):
            off = base + ci * chunk
            pltpu.sync_copy(idx_hbm.at[pl.ds(off, chunk)], idx_v)

            def inner(k, c2):
                iv = idx_v[pl.ds(k * 16, 16)]
                for t in range(num_tables):
                    out_v[t][pl.ds(k * 16, 16)] = plsc.load_gather(
                        tbl_v[t], [iv])
                return c2

            lax.fori_loop(0, chunk // 16, inner, 0)
            for t in range(num_tables):
                pltpu.sync_copy(out_v[t], out_hbm[t].at[pl.ds(off, chunk)])
            return carry

        lax.fori_loop(0, n_chunks, outer, 0)

    return pl.kernel(
        body,
        out_type=[jax.ShapeDtypeStruct((total,), jnp.float32)] * num_tables,
        mesh=mesh,
        compiler_params=pltpu.CompilerParams(needs_layout_passes=False),
        scratch_types=(
            [pltpu.VMEM((N,), jnp.float32)] * num_tables
            + [pltpu.VMEM((chunk,), jnp.int32)]
            + [pltpu.VMEM((chunk,), jnp.float32)] * num_tables
        ),
    )


def _make_sc_gather_rows(total, depth):
    per_w = total // SC_WORKERS
    chunk = _pick_chunk(per_w, 128)
    n_chunks = per_w // chunk
    mesh = plsc.VectorSubcoreMesh(core_axis_name="c", subcore_axis_name="s")

    def body(tbl_hbm, idx_hbm, out_hbm, idx_v, rows_v, sem):
        wid = lax.axis_index("s") * 2 + lax.axis_index("c")
        base = wid * per_w

        def outer(ci, carry):
            off = base + ci * chunk
            pltpu.sync_copy(idx_hbm.at[pl.ds(off, chunk)], idx_v)
            pltpu.async_copy(tbl_hbm.at[idx_v], rows_v, sem).wait()
            pltpu.sync_copy(rows_v, out_hbm.at[pl.ds(off, chunk)])
            return carry

        lax.fori_loop(0, n_chunks, outer, 0)

    return pl.kernel(
        body,
        out_type=jax.ShapeDtypeStruct((total, depth), jnp.float32),
        mesh=mesh,
        compiler_params=pltpu.CompilerParams(needs_layout_passes=False),
        scratch_types=[
            pltpu.VMEM((chunk,), jnp.int32),
            pltpu.VMEM((chunk, depth), jnp.float32),
            pltpu.SemaphoreType.DMA,
        ],
    )


def _angle_parts(ax, ay, az, bx, by, bz):
    cx = ay * bz - az * by
    cy = az * bx - ax * bz
    cz = ax * by - ay * bx
    cn = jnp.sqrt(cx * cx + cy * cy + cz * cz + 1e-12)
    dot = ax * bx + ay * by + az * bz
    return jnp.arctan2(cn, dot)


def _attn_body(npos_ref, nnorm_ref, cpx_ref, cpy_ref, cpz_ref,
               cnx_ref, cny_ref, cnz_ref, xq_ref, cx_ref,
               Wq_ref, Wk_ref, Wv_ref, Wp1_ref, Wp2_ref, Wo_ref, out_ref):
    B = ATT_BLOCK
    npos = npos_ref[...]
    nnorm = nnorm_ref[...]
    qpx, qpy, qpz = npos[:, 0:1], npos[:, 1:2], npos[:, 2:3]
    nx, ny, nz = nnorm[:, 0:1], nnorm[:, 1:2], nnorm[:, 2:3]
    cpx, cpy, cpz = cpx_ref[...], cpy_ref[...], cpz_ref[...]
    cnx, cny, cnz = cnx_ref[...], cny_ref[...], cnz_ref[...]

    dx, dy, dz = cpx - qpx, cpy - qpy, cpz - qpz
    dist = jnp.sqrt(dx * dx + dy * dy + dz * dz + 1e-12)
    f1 = _angle_parts(nx, ny, nz, dx, dy, dz)
    f2 = _angle_parts(cnx, cny, cnz, dx, dy, dz)
    f3 = _angle_parts(nx, ny, nz, cnx, cny, cnz)

    Wp1 = Wp1_ref[...]
    pe3 = (f1[:, :, None] * Wp1[0:1, :][None]
           + f2[:, :, None] * Wp1[1:2, :][None]
           + f3[:, :, None] * Wp1[2:3, :][None]
           + dist[:, :, None] * Wp1[3:4, :][None])
    pe2 = jnp.maximum(pe3, 0.0).reshape(B * NSAMPLE, HIDDEN) @ Wp2_ref[...]

    cx = cx_ref[...]
    k2 = cx @ Wk_ref[...] + pe2
    v2 = cx @ Wv_ref[...] + pe2
    q2 = xq_ref[...] @ Wq_ref[...]

    k3 = k2.reshape(B, NSAMPLE, HIDDEN)
    v3 = v2.reshape(B, NSAMPLE, HIDDEN)
    prod = q2[:, None, :] * k3
    scale = 1.0 / jnp.sqrt(jnp.float32(HEAD_DIM))
    outs = []
    for h in range(NUM_HEADS):
        sl = slice(h * HEAD_DIM, (h + 1) * HEAD_DIM)
        lh = jnp.sum(prod[:, :, sl], axis=2) * scale
        mh = jnp.max(lh, axis=1, keepdims=True)
        e = jnp.exp(lh - mh)
        a = e / jnp.sum(e, axis=1, keepdims=True)
        outs.append(jnp.sum(a[:, :, None] * v3[:, :, sl], axis=1))
    out = jnp.concatenate(outs, axis=1)
    out_ref[...] = out @ Wo_ref[...]


def _attention(npos, nnorm, cpx, cpy, cpz, cnx, cny, cnz, xq, cx,
               Wq, Wk, Wv, Wp1, Wp2, Wo):
    B = ATT_BLOCK
    bs = lambda r: pl.BlockSpec((B, r), lambda i: (i, 0))
    full = lambda a, b: pl.BlockSpec((a, b), lambda i: (0, 0))
    return pl.pallas_call(
        _attn_body,
        grid=(M // B,),
        in_specs=[
            bs(3), bs(3),
            bs(NSAMPLE), bs(NSAMPLE), bs(NSAMPLE),
            bs(NSAMPLE), bs(NSAMPLE), bs(NSAMPLE),
            bs(IN_PLANES),
            pl.BlockSpec((B * NSAMPLE, IN_PLANES), lambda i: (i, 0)),
            full(IN_PLANES, HIDDEN), full(IN_PLANES, HIDDEN),
            full(IN_PLANES, HIDDEN), full(4, HIDDEN),
            full(HIDDEN, HIDDEN), full(HIDDEN, HIDDEN),
        ],
        out_specs=pl.BlockSpec((B, HIDDEN), lambda i: (i, 0)),
        out_shape=jax.ShapeDtypeStruct((M, HIDDEN), jnp.float32),
        compiler_params=pltpu.CompilerParams(
            vmem_limit_bytes=60 * 1024 * 1024),
    )(npos, nnorm, cpx, cpy, cpz, cnx, cny, cnz, xq, cx,
      Wq, Wk, Wv, Wp1, Wp2, Wo)


def kernel(p, x, o, n, Wq, Wk, Wv, Wp1, Wp2, Wo):
    idx = _grid_sampling(p, M).astype(jnp.int32)
    n_p = p[idx]
    n_n = n[idx]
    n_o = (o // STRIDE).astype(jnp.int32)

    group_idx = _knn(n_p, p.T)
    gi = group_idx.reshape(-1)

    g6 = _make_sc_gather_cols(6, M * NSAMPLE)
    gpx, gpy, gpz, gnx, gny, gnz = g6(
        p[:, 0], p[:, 1], p[:, 2], n[:, 0], n[:, 1], n[:, 2], gi)
    cpx = gpx.reshape(M, NSAMPLE)
    cpy = gpy.reshape(M, NSAMPLE)
    cpz = gpz.reshape(M, NSAMPLE)
    cnx = gnx.reshape(M, NSAMPLE)
    cny = gny.reshape(M, NSAMPLE)
    cnz = gnz.reshape(M, NSAMPLE)

    gi_all = jnp.concatenate([gi, idx])
    rows = _make_sc_gather_rows(M * NSAMPLE + M, IN_PLANES)(x, gi_all)
    cx = rows[:M * NSAMPLE]
    xq = rows[M * NSAMPLE:]

    x_out = _attention(n_p, n_n, cpx, cpy, cpz, cnx, cny, cnz, xq, cx,
                       Wq, Wk, Wv, Wp1, Wp2, Wo)
    return (n_p, x_out, n_o, n_n, idx)
